# baseline (speedup 1.0000x reference)
import functools

import jax
import jax.numpy as jnp
from jax import lax
from jax.experimental import pallas as pl
from jax.experimental.pallas import tpu as pltpu
from jax.experimental.pallas import tpu_sc as plsc

B = 4096
E = 131072
NC = 2
NS = 16
L = 16
NW = NC * NS
EPW = E // NW
ROWS_PW = EPW // 128


_CONV_BLK = 128


def _conv_body(cw_ref, cb_ref, edge_ref, x_ref, out_ref, src_ref, dst_ref,
               sem):
    @pl.when(pl.program_id(0) == 0)
    def _():
        cp0 = pltpu.make_async_copy(edge_ref.at[0], src_ref, sem)
        cp1 = pltpu.make_async_copy(edge_ref.at[1], dst_ref, sem)
        cp0.start()
        cp1.start()
        cp0.wait()
        cp1.wait()

    x = x_ref[...]
    w00 = cw_ref[0, 0]
    w01 = cw_ref[0, 1]
    w10 = cw_ref[0, 2]
    w11 = cw_ref[0, 3]
    xd = pltpu.roll(x, 63, 1)
    a = w00 * x + w10 * xd + cb_ref[0, 0]
    b = w01 * x + w11 * xd
    r0 = lax.broadcasted_iota(jnp.int32, (64, 64), 0)
    c0 = lax.broadcasted_iota(jnp.int32, (64, 64), 1)
    shift = (r0 == c0 + 1).astype(jnp.float32)
    bs = lax.dot_general(
        b.reshape(_CONV_BLK * 64, 64), shift, (((1,), (0,)), ((), ())),
        preferred_element_type=jnp.float32,
    ).reshape(_CONV_BLK, 64, 64)
    m = a + bs
    s = 1.0 / (1.0 + jnp.exp2(m))
    mx = (lax.broadcasted_iota(jnp.int32, (1, 1, 64), 2) < 63).astype(
        jnp.float32)
    sm = s * mx
    t = jnp.sum(sm, axis=(1, 2)) - jnp.sum(sm[:, 63, :], axis=1)
    out_ref[...] = (t * (1.0 / 3969.0)).reshape(1, _CONV_BLK)


def _conv_feats(data3, cw, cb, edge_index):
    nblk = B // _CONV_BLK
    return pl.pallas_call(
        _conv_body,
        grid=(nblk,),
        in_specs=[
            pl.BlockSpec(memory_space=pltpu.SMEM),
            pl.BlockSpec(memory_space=pltpu.SMEM),
            pl.BlockSpec(memory_space=pl.ANY),
            pl.BlockSpec((_CONV_BLK, 64, 64), lambda i: (i, 0, 0)),
        ],
        out_specs=(
            pl.BlockSpec((1, _CONV_BLK), lambda i: (0, i)),
            pl.BlockSpec(memory_space=pl.ANY),
            pl.BlockSpec(memory_space=pl.ANY),
        ),
        out_shape=(
            jax.ShapeDtypeStruct((1, B), jnp.float32),
            jax.ShapeDtypeStruct((E,), jnp.int32),
            jax.ShapeDtypeStruct((E,), jnp.int32),
        ),
        scratch_shapes=[pltpu.SemaphoreType.DMA],
    )(cw, cb, edge_index, data3)


_MV_BLK = 512


def _mv_body(x_ref, w_ref, b_ref, out_ref):
    out_ref[...] = lax.dot_general(
        x_ref[...], w_ref[...], (((1,), (1,)), ((), ())),
        preferred_element_type=jnp.float32,
    ) + b_ref[...]


def _matvec(x2d, W, b2d):
    nblk = B // _MV_BLK
    return pl.pallas_call(
        _mv_body,
        grid=(nblk,),
        in_specs=[
            pl.BlockSpec((1, B), lambda i: (0, 0)),
            pl.BlockSpec((_MV_BLK, B), lambda i: (i, 0)),
            pl.BlockSpec((1, _MV_BLK), lambda i: (0, i)),
        ],
        out_specs=pl.BlockSpec((1, _MV_BLK), lambda i: (0, i)),
        out_shape=jax.ShapeDtypeStruct((1, B), jnp.float32),
    )(x2d, W, b2d)


def _sc_segsum_body(x_hbm, src_hbm, dst_hbm, out_hbm,
                    x_v, src_v, dst1_v, dst_v, vals_v, zero_v, acc_sh, sem):
    c = lax.axis_index("c")
    s = lax.axis_index("s")
    chunk = c * NS + s

    def zbody(i, _):
        zero_v[pl.ds(i * L, L)] = jnp.zeros((L,), jnp.float32)
        return 0
    lax.fori_loop(0, (B // NS) // L, zbody, 0)
    pltpu.sync_copy(zero_v, acc_sh.at[pl.ds(s * (B // NS), B // NS)])

    pltpu.sync_copy(x_hbm, x_v)
    pltpu.sync_copy(src_hbm.at[pl.ds(chunk * EPW, EPW)], src_v)
    pltpu.sync_copy(dst_hbm.at[pl.ds(chunk * EPW, EPW)], dst1_v)

    def gbody(i, _):
        j = i // (128 // L)
        o = (i % (128 // L)) * L
        idx = src_v[pl.ds(i * L, L)]
        vals_v[j, pl.ds(o, L)] = plsc.load_gather(x_v, [idx])
        dst_v[j, pl.ds(o, L)] = dst1_v[pl.ds(i * L, L)]
        return 0
    lax.fori_loop(0, EPW // L, gbody, 0)

    plsc.subcore_barrier()

    copies = []
    for j in range(ROWS_PW):
        copies.append(
            pltpu.async_copy(vals_v.at[j], acc_sh.at[dst_v.at[j]], sem,
                             add=True))
    for cp in copies:
        cp.wait()

    plsc.subcore_barrier()

    @pl.when(s == 0)
    def _():
        pltpu.sync_copy(acc_sh, out_hbm.at[c])


@functools.lru_cache(maxsize=None)
def _make_sc_segsum():
    mesh = plsc.VectorSubcoreMesh(
        core_axis_name="c", subcore_axis_name="s",
        num_cores=NC, num_subcores=NS)
    return pl.kernel(
        _sc_segsum_body,
        out_type=jax.ShapeDtypeStruct((NC, B), jnp.float32),
        mesh=mesh,
        compiler_params=pltpu.CompilerParams(needs_layout_passes=False),
        scratch_types=[
            pltpu.VMEM((B,), jnp.float32),
            pltpu.VMEM((EPW,), jnp.int32),
            pltpu.VMEM((EPW,), jnp.int32),
            pltpu.VMEM((ROWS_PW, 128), jnp.int32),
            pltpu.VMEM((ROWS_PW, 128), jnp.float32),
            pltpu.VMEM((B // NS,), jnp.float32),
            pltpu.VMEM_SHARED((B,), jnp.float32),
            pltpu.SemaphoreType.DMA,
        ],
    )


def _sc_segsum(x, src1d, dst1d):
    return _make_sc_segsum()(x, src1d, dst1d)


def _nm(p_ref, d_ref):
    sums = p_ref[0:1, :] + p_ref[1:2, :]
    deg = d_ref[0:1, :] + d_ref[1:2, :]
    return jnp.where(deg > 0, sums / jnp.maximum(deg, 1.0), 0.0)


def _mv2_body(x_ref, p_ref, d_ref, w_ref, b_ref, cur2_ref, out_ref):
    cur2 = jnp.tanh(x_ref[...] + _nm(p_ref, d_ref))
    cur2_ref[...] = cur2
    out_ref[...] = lax.dot_general(
        cur2, w_ref[...], (((1,), (1,)), ((), ())),
        preferred_element_type=jnp.float32,
    ) + b_ref[...]


def _mv2(x2d, p, d, W, b2d):
    nblk = B // _MV_BLK
    return pl.pallas_call(
        _mv2_body,
        grid=(nblk,),
        in_specs=[
            pl.BlockSpec((1, B), lambda i: (0, 0)),
            pl.BlockSpec((NC, B), lambda i: (0, 0)),
            pl.BlockSpec((NC, B), lambda i: (0, 0)),
            pl.BlockSpec((_MV_BLK, B), lambda i: (i, 0)),
            pl.BlockSpec((1, _MV_BLK), lambda i: (0, i)),
        ],
        out_specs=(
            pl.BlockSpec((1, B), lambda i: (0, 0)),
            pl.BlockSpec((1, _MV_BLK), lambda i: (0, i)),
        ),
        out_shape=(
            jax.ShapeDtypeStruct((1, B), jnp.float32),
            jax.ShapeDtypeStruct((1, B), jnp.float32),
        ),
    )(x2d, p, d, W, b2d)


def _final_body(b3_ref, x_ref, p_ref, d_ref, w3_ref, cur4_ref, out_ref):
    cur4 = jnp.tanh(x_ref[...] + _nm(p_ref, d_ref))
    cur4_ref[...] = cur4
    out_ref[0, 0] = jnp.sum(cur4 * w3_ref[...]) + b3_ref[0, 0]


def _final(x2d, p, d, W3, b3_2d):
    return pl.pallas_call(
        _final_body,
        in_specs=[
            pl.BlockSpec(memory_space=pltpu.SMEM),
            pl.BlockSpec((1, B), lambda: (0, 0)),
            pl.BlockSpec((NC, B), lambda: (0, 0)),
            pl.BlockSpec((NC, B), lambda: (0, 0)),
            pl.BlockSpec((1, B), lambda: (0, 0)),
        ],
        out_specs=(
            pl.BlockSpec((1, B), lambda: (0, 0)),
            pl.BlockSpec(memory_space=pltpu.SMEM),
        ),
        out_shape=(
            jax.ShapeDtypeStruct((1, B), jnp.float32),
            jax.ShapeDtypeStruct((1, 1), jnp.float32),
        ),
    )(b3_2d, x2d, p, d, W3)


def kernel(data, edge_index, conv_w, conv_b, W1, b1, W2, b2, W3, b3):
    data3 = data.reshape(B, 64, 64)
    neg_log2e = -1.4426950408889634
    cw = conv_w.reshape(1, 4) * neg_log2e
    cb = conv_b.reshape(1, 1) * neg_log2e
    ones = jnp.ones((B,), jnp.float32)

    feats2d, src1d, dst1d = _conv_feats(data3, cw, cb, edge_index)
    degp = _sc_segsum(ones, src1d, dst1d)
    cur1_2d = _matvec(feats2d, W1, b1.reshape(1, B))

    p1 = _sc_segsum(cur1_2d.reshape(B), src1d, dst1d)
    cur2_2d, cur3_2d = _mv2(cur1_2d, p1, degp, W2, b2.reshape(1, B))

    p2 = _sc_segsum(cur3_2d.reshape(B), src1d, dst1d)
    cur4_2d, out5 = _final(cur3_2d, p2, degp, W3.reshape(1, B),
                           b3.reshape(1, 1))

    return (feats2d.reshape(B), cur1_2d.reshape(B), cur2_2d.reshape(B),
            cur3_2d.reshape(B), cur4_2d.reshape(B), out5.reshape(1))


# speedup vs baseline: 1.1975x; 1.1975x over previous
import functools

import jax
import jax.numpy as jnp
from jax import lax
from jax.experimental import pallas as pl
from jax.experimental.pallas import tpu as pltpu
from jax.experimental.pallas import tpu_sc as plsc

B = 4096
E = 131072
NC = 2
NS = 16
L = 16
NW = NC * NS
EPW = E // NW
ROWS_PW = EPW // 128


_CONV_BLK = 128


def _edge_prep_body(edge_ref, src_ref, dst_ref, sem):
    cp0 = pltpu.make_async_copy(edge_ref.at[0], src_ref, sem)
    cp1 = pltpu.make_async_copy(edge_ref.at[1], dst_ref, sem)
    cp0.start()
    cp1.start()
    cp0.wait()
    cp1.wait()


def _edge_prep(edge_index):
    return pl.pallas_call(
        _edge_prep_body,
        in_specs=[pl.BlockSpec(memory_space=pl.ANY)],
        out_specs=(
            pl.BlockSpec(memory_space=pl.ANY),
            pl.BlockSpec(memory_space=pl.ANY),
        ),
        out_shape=(
            jax.ShapeDtypeStruct((E,), jnp.int32),
            jax.ShapeDtypeStruct((E,), jnp.int32),
        ),
        scratch_shapes=[pltpu.SemaphoreType.DMA],
    )(edge_index)


def _conv_body(cw_ref, cb_ref, x_ref, out_ref):
    x = x_ref[...]
    w00 = cw_ref[0, 0]
    w01 = cw_ref[0, 1]
    w10 = cw_ref[0, 2]
    w11 = cw_ref[0, 3]
    xd = pltpu.roll(x, 63, 1)
    a = w00 * x + w10 * xd + cb_ref[0, 0]
    b = w01 * x + w11 * xd
    r0 = lax.broadcasted_iota(jnp.int32, (64, 64), 0)
    c0 = lax.broadcasted_iota(jnp.int32, (64, 64), 1)
    shift = (r0 == c0 + 1).astype(jnp.float32)
    bs = lax.dot_general(
        b.reshape(_CONV_BLK * 64, 64), shift, (((1,), (0,)), ((), ())),
        preferred_element_type=jnp.float32,
    ).reshape(_CONV_BLK, 64, 64)
    m = a + bs
    s = 1.0 / (1.0 + jnp.exp2(m))
    mx = (lax.broadcasted_iota(jnp.int32, (1, 1, 64), 2) < 63).astype(
        jnp.float32)
    sm = s * mx
    t = jnp.sum(sm, axis=(1, 2)) - jnp.sum(sm[:, 63, :], axis=1)
    out_ref[...] = (t * (1.0 / 3969.0)).reshape(1, _CONV_BLK)


def _conv_feats(data3, cw, cb):
    nblk = B // _CONV_BLK
    return pl.pallas_call(
        _conv_body,
        grid=(nblk,),
        in_specs=[
            pl.BlockSpec(memory_space=pltpu.SMEM),
            pl.BlockSpec(memory_space=pltpu.SMEM),
            pl.BlockSpec((_CONV_BLK, 64, 64), lambda i: (i, 0, 0)),
        ],
        out_specs=pl.BlockSpec((1, _CONV_BLK), lambda i: (0, i)),
        out_shape=jax.ShapeDtypeStruct((1, B), jnp.float32),
    )(cw, cb, data3)


_MV_BLK = 512


def _mv_body(x_ref, w_ref, b_ref, out_ref):
    t = jnp.sum(w_ref[...] * x_ref[...], axis=1)
    out_ref[...] = t.reshape(1, _MV_BLK) + b_ref[...]


def _matvec(x2d, W, b2d):
    nblk = B // _MV_BLK
    return pl.pallas_call(
        _mv_body,
        grid=(nblk,),
        in_specs=[
            pl.BlockSpec((1, B), lambda i: (0, 0)),
            pl.BlockSpec((_MV_BLK, B), lambda i: (i, 0)),
            pl.BlockSpec((1, _MV_BLK), lambda i: (0, i)),
        ],
        out_specs=pl.BlockSpec((1, _MV_BLK), lambda i: (0, i)),
        out_shape=jax.ShapeDtypeStruct((1, B), jnp.float32),
    )(x2d, W, b2d)


def _sc_segsum_body(x_hbm, src_hbm, dst_hbm, out_hbm,
                    x_v, src_v, dst1_v, dst_v, vals_v, zero_v, acc_sh, sem):
    c = lax.axis_index("c")
    s = lax.axis_index("s")
    chunk = c * NS + s

    def zbody(i, _):
        zero_v[pl.ds(i * L, L)] = jnp.zeros((L,), jnp.float32)
        return 0
    lax.fori_loop(0, (B // NS) // L, zbody, 0)
    pltpu.sync_copy(zero_v, acc_sh.at[pl.ds(s * (B // NS), B // NS)])

    pltpu.sync_copy(x_hbm, x_v)
    pltpu.sync_copy(src_hbm.at[pl.ds(chunk * EPW, EPW)], src_v)
    pltpu.sync_copy(dst_hbm.at[pl.ds(chunk * EPW, EPW)], dst1_v)

    def gbody(i, _):
        j = i // (128 // L)
        o = (i % (128 // L)) * L
        idx = src_v[pl.ds(i * L, L)]
        vals_v[j, pl.ds(o, L)] = plsc.load_gather(x_v, [idx])
        dst_v[j, pl.ds(o, L)] = dst1_v[pl.ds(i * L, L)]
        return 0
    lax.fori_loop(0, EPW // L, gbody, 0)

    plsc.subcore_barrier()

    copies = []
    for j in range(ROWS_PW):
        copies.append(
            pltpu.async_copy(vals_v.at[j], acc_sh.at[dst_v.at[j]], sem,
                             add=True))
    for cp in copies:
        cp.wait()

    plsc.subcore_barrier()

    @pl.when(s == 0)
    def _():
        pltpu.sync_copy(acc_sh, out_hbm.at[c])


@functools.lru_cache(maxsize=None)
def _make_sc_segsum():
    mesh = plsc.VectorSubcoreMesh(
        core_axis_name="c", subcore_axis_name="s",
        num_cores=NC, num_subcores=NS)
    return pl.kernel(
        _sc_segsum_body,
        out_type=jax.ShapeDtypeStruct((NC, B), jnp.float32),
        mesh=mesh,
        compiler_params=pltpu.CompilerParams(needs_layout_passes=False),
        scratch_types=[
            pltpu.VMEM((B,), jnp.float32),
            pltpu.VMEM((EPW,), jnp.int32),
            pltpu.VMEM((EPW,), jnp.int32),
            pltpu.VMEM((ROWS_PW, 128), jnp.int32),
            pltpu.VMEM((ROWS_PW, 128), jnp.float32),
            pltpu.VMEM((B // NS,), jnp.float32),
            pltpu.VMEM_SHARED((B,), jnp.float32),
            pltpu.SemaphoreType.DMA,
        ],
    )


def _sc_segsum(x, src1d, dst1d):
    return _make_sc_segsum()(x, src1d, dst1d)


def _nm(p_ref, d_ref):
    sums = p_ref[0:1, :] + p_ref[1:2, :]
    deg = d_ref[0:1, :] + d_ref[1:2, :]
    return jnp.where(deg > 0, sums / jnp.maximum(deg, 1.0), 0.0)


def _mv2_body(x_ref, p_ref, d_ref, w_ref, b_ref, cur2_ref, out_ref):
    cur2 = jnp.tanh(x_ref[...] + _nm(p_ref, d_ref))
    cur2_ref[...] = cur2
    t = jnp.sum(w_ref[...] * cur2, axis=1)
    out_ref[...] = t.reshape(1, _MV_BLK) + b_ref[...]


def _mv2(x2d, p, d, W, b2d):
    nblk = B // _MV_BLK
    return pl.pallas_call(
        _mv2_body,
        grid=(nblk,),
        in_specs=[
            pl.BlockSpec((1, B), lambda i: (0, 0)),
            pl.BlockSpec((NC, B), lambda i: (0, 0)),
            pl.BlockSpec((NC, B), lambda i: (0, 0)),
            pl.BlockSpec((_MV_BLK, B), lambda i: (i, 0)),
            pl.BlockSpec((1, _MV_BLK), lambda i: (0, i)),
        ],
        out_specs=(
            pl.BlockSpec((1, B), lambda i: (0, 0)),
            pl.BlockSpec((1, _MV_BLK), lambda i: (0, i)),
        ),
        out_shape=(
            jax.ShapeDtypeStruct((1, B), jnp.float32),
            jax.ShapeDtypeStruct((1, B), jnp.float32),
        ),
    )(x2d, p, d, W, b2d)


def _final_body(b3_ref, x_ref, p_ref, d_ref, w3_ref, cur4_ref, out_ref):
    cur4 = jnp.tanh(x_ref[...] + _nm(p_ref, d_ref))
    cur4_ref[...] = cur4
    out_ref[0, 0] = jnp.sum(cur4 * w3_ref[...]) + b3_ref[0, 0]


def _final(x2d, p, d, W3, b3_2d):
    return pl.pallas_call(
        _final_body,
        in_specs=[
            pl.BlockSpec(memory_space=pltpu.SMEM),
            pl.BlockSpec((1, B), lambda: (0, 0)),
            pl.BlockSpec((NC, B), lambda: (0, 0)),
            pl.BlockSpec((NC, B), lambda: (0, 0)),
            pl.BlockSpec((1, B), lambda: (0, 0)),
        ],
        out_specs=(
            pl.BlockSpec((1, B), lambda: (0, 0)),
            pl.BlockSpec(memory_space=pltpu.SMEM),
        ),
        out_shape=(
            jax.ShapeDtypeStruct((1, B), jnp.float32),
            jax.ShapeDtypeStruct((1, 1), jnp.float32),
        ),
    )(b3_2d, x2d, p, d, W3)


def kernel(data, edge_index, conv_w, conv_b, W1, b1, W2, b2, W3, b3):
    data3 = data.reshape(B, 64, 64)
    neg_log2e = -1.4426950408889634
    cw = conv_w.reshape(1, 4) * neg_log2e
    cb = conv_b.reshape(1, 1) * neg_log2e
    ones = jnp.ones((B,), jnp.float32)

    src1d, dst1d = _edge_prep(edge_index)
    degp = _sc_segsum(ones, src1d, dst1d)
    feats2d = _conv_feats(data3, cw, cb)
    cur1_2d = _matvec(feats2d, W1, b1.reshape(1, B))

    p1 = _sc_segsum(cur1_2d.reshape(B), src1d, dst1d)
    cur2_2d, cur3_2d = _mv2(cur1_2d, p1, degp, W2, b2.reshape(1, B))

    p2 = _sc_segsum(cur3_2d.reshape(B), src1d, dst1d)
    cur4_2d, out5 = _final(cur3_2d, p2, degp, W3.reshape(1, B),
                           b3.reshape(1, 1))

    return (feats2d.reshape(B), cur1_2d.reshape(B), cur2_2d.reshape(B),
            cur3_2d.reshape(B), cur4_2d.reshape(B), out5.reshape(1))


# speedup vs baseline: 1.2030x; 1.0046x over previous
import functools

import jax
import jax.numpy as jnp
from jax import lax
from jax.experimental import pallas as pl
from jax.experimental.pallas import tpu as pltpu
from jax.experimental.pallas import tpu_sc as plsc

B = 4096
E = 131072
NC = 2
NS = 16
L = 16
NW = NC * NS
EPW = E // NW
ROWS_PW = EPW // 128


_CONV_BLK = 128


def _conv_body(cw_ref, cb_ref, x_ref, out_ref):
    x = x_ref[...]
    w00 = cw_ref[0, 0]
    w01 = cw_ref[0, 1]
    w10 = cw_ref[0, 2]
    w11 = cw_ref[0, 3]
    xd = pltpu.roll(x, 63, 1)
    a = w00 * x + w10 * xd + cb_ref[0, 0]
    b = w01 * x + w11 * xd
    r0 = lax.broadcasted_iota(jnp.int32, (64, 64), 0)
    c0 = lax.broadcasted_iota(jnp.int32, (64, 64), 1)
    shift = (r0 == c0 + 1).astype(jnp.float32)
    bs = lax.dot_general(
        b.reshape(_CONV_BLK * 64, 64), shift, (((1,), (0,)), ((), ())),
        preferred_element_type=jnp.float32,
    ).reshape(_CONV_BLK, 64, 64)
    m = a + bs
    s = 1.0 / (1.0 + jnp.exp2(m))
    mx = (lax.broadcasted_iota(jnp.int32, (1, 1, 64), 2) < 63).astype(
        jnp.float32)
    sm = s * mx
    t = jnp.sum(sm, axis=(1, 2)) - jnp.sum(sm[:, 63, :], axis=1)
    out_ref[...] = (t * (1.0 / 3969.0)).reshape(1, _CONV_BLK)


def _conv_feats(data3, cw, cb):
    nblk = B // _CONV_BLK
    return pl.pallas_call(
        _conv_body,
        grid=(nblk,),
        in_specs=[
            pl.BlockSpec(memory_space=pltpu.SMEM),
            pl.BlockSpec(memory_space=pltpu.SMEM),
            pl.BlockSpec((_CONV_BLK, 64, 64), lambda i: (i, 0, 0)),
        ],
        out_specs=pl.BlockSpec((1, _CONV_BLK), lambda i: (0, i)),
        out_shape=jax.ShapeDtypeStruct((1, B), jnp.float32),
    )(cw, cb, data3)


_MV_BLK = 512


def _mv_body(x_ref, w_ref, b_ref, out_ref):
    t = jnp.sum(w_ref[...] * x_ref[...], axis=1)
    out_ref[...] = t.reshape(1, _MV_BLK) + b_ref[...]


def _matvec(x2d, W, b2d):
    nblk = B // _MV_BLK
    return pl.pallas_call(
        _mv_body,
        grid=(nblk,),
        in_specs=[
            pl.BlockSpec((1, B), lambda i: (0, 0)),
            pl.BlockSpec((_MV_BLK, B), lambda i: (i, 0)),
            pl.BlockSpec((1, _MV_BLK), lambda i: (0, i)),
        ],
        out_specs=pl.BlockSpec((1, _MV_BLK), lambda i: (0, i)),
        out_shape=jax.ShapeDtypeStruct((1, B), jnp.float32),
    )(x2d, W, b2d)


def _sc_segsum_body(x_hbm, src_hbm, dst_hbm, out_hbm,
                    x_v, src_v, dst1_v, dst_v, vals_v, zero_v, acc_sh, sem):
    c = lax.axis_index("c")
    s = lax.axis_index("s")
    chunk = c * NS + s

    def zbody(i, _):
        zero_v[pl.ds(i * L, L)] = jnp.zeros((L,), jnp.float32)
        return 0
    lax.fori_loop(0, (B // NS) // L, zbody, 0)
    pltpu.sync_copy(zero_v, acc_sh.at[pl.ds(s * (B // NS), B // NS)])

    pltpu.sync_copy(x_hbm, x_v)
    pltpu.sync_copy(src_hbm.at[pl.ds(chunk * EPW, EPW)], src_v)
    pltpu.sync_copy(dst_hbm.at[pl.ds(chunk * EPW, EPW)], dst1_v)

    def gbody(i, _):
        j = i // (128 // L)
        o = (i % (128 // L)) * L
        idx = src_v[pl.ds(i * L, L)]
        vals_v[j, pl.ds(o, L)] = plsc.load_gather(x_v, [idx])
        dst_v[j, pl.ds(o, L)] = dst1_v[pl.ds(i * L, L)]
        return 0
    lax.fori_loop(0, EPW // L, gbody, 0)

    plsc.subcore_barrier()

    copies = []
    for j in range(ROWS_PW):
        copies.append(
            pltpu.async_copy(vals_v.at[j], acc_sh.at[dst_v.at[j]], sem,
                             add=True))
    for cp in copies:
        cp.wait()

    plsc.subcore_barrier()

    @pl.when(s == 0)
    def _():
        pltpu.sync_copy(acc_sh, out_hbm.at[c])


@functools.lru_cache(maxsize=None)
def _make_sc_segsum():
    mesh = plsc.VectorSubcoreMesh(
        core_axis_name="c", subcore_axis_name="s",
        num_cores=NC, num_subcores=NS)
    return pl.kernel(
        _sc_segsum_body,
        out_type=jax.ShapeDtypeStruct((NC, B), jnp.float32),
        mesh=mesh,
        compiler_params=pltpu.CompilerParams(needs_layout_passes=False),
        scratch_types=[
            pltpu.VMEM((B,), jnp.float32),
            pltpu.VMEM((EPW,), jnp.int32),
            pltpu.VMEM((EPW,), jnp.int32),
            pltpu.VMEM((ROWS_PW, 128), jnp.int32),
            pltpu.VMEM((ROWS_PW, 128), jnp.float32),
            pltpu.VMEM((B // NS,), jnp.float32),
            pltpu.VMEM_SHARED((B,), jnp.float32),
            pltpu.SemaphoreType.DMA,
        ],
    )


def _sc_segsum(x, src1d, dst1d):
    return _make_sc_segsum()(x, src1d, dst1d)


def _nm(p_ref, d_ref):
    sums = p_ref[0:1, :] + p_ref[1:2, :]
    deg = d_ref[0:1, :] + d_ref[1:2, :]
    return jnp.where(deg > 0, sums / jnp.maximum(deg, 1.0), 0.0)


def _mv2_body(x_ref, p_ref, d_ref, w_ref, b_ref, cur2_ref, out_ref):
    cur2 = jnp.tanh(x_ref[...] + _nm(p_ref, d_ref))
    cur2_ref[...] = cur2
    t = jnp.sum(w_ref[...] * cur2, axis=1)
    out_ref[...] = t.reshape(1, _MV_BLK) + b_ref[...]


def _mv2(x2d, p, d, W, b2d):
    nblk = B // _MV_BLK
    return pl.pallas_call(
        _mv2_body,
        grid=(nblk,),
        in_specs=[
            pl.BlockSpec((1, B), lambda i: (0, 0)),
            pl.BlockSpec((NC, B), lambda i: (0, 0)),
            pl.BlockSpec((NC, B), lambda i: (0, 0)),
            pl.BlockSpec((_MV_BLK, B), lambda i: (i, 0)),
            pl.BlockSpec((1, _MV_BLK), lambda i: (0, i)),
        ],
        out_specs=(
            pl.BlockSpec((1, B), lambda i: (0, 0)),
            pl.BlockSpec((1, _MV_BLK), lambda i: (0, i)),
        ),
        out_shape=(
            jax.ShapeDtypeStruct((1, B), jnp.float32),
            jax.ShapeDtypeStruct((1, B), jnp.float32),
        ),
    )(x2d, p, d, W, b2d)


def _final_body(b3_ref, x_ref, p_ref, d_ref, w3_ref, cur4_ref, out_ref):
    cur4 = jnp.tanh(x_ref[...] + _nm(p_ref, d_ref))
    cur4_ref[...] = cur4
    out_ref[0, 0] = jnp.sum(cur4 * w3_ref[...]) + b3_ref[0, 0]


def _final(x2d, p, d, W3, b3_2d):
    return pl.pallas_call(
        _final_body,
        in_specs=[
            pl.BlockSpec(memory_space=pltpu.SMEM),
            pl.BlockSpec((1, B), lambda: (0, 0)),
            pl.BlockSpec((NC, B), lambda: (0, 0)),
            pl.BlockSpec((NC, B), lambda: (0, 0)),
            pl.BlockSpec((1, B), lambda: (0, 0)),
        ],
        out_specs=(
            pl.BlockSpec((1, B), lambda: (0, 0)),
            pl.BlockSpec(memory_space=pltpu.SMEM),
        ),
        out_shape=(
            jax.ShapeDtypeStruct((1, B), jnp.float32),
            jax.ShapeDtypeStruct((1, 1), jnp.float32),
        ),
    )(b3_2d, x2d, p, d, W3)


def kernel(data, edge_index, conv_w, conv_b, W1, b1, W2, b2, W3, b3):
    data3 = data.reshape(B, 64, 64)
    neg_log2e = -1.4426950408889634
    cw = conv_w.reshape(1, 4) * neg_log2e
    cb = conv_b.reshape(1, 1) * neg_log2e
    ones = jnp.ones((B,), jnp.float32)

    src1d = jnp.minimum(edge_index[0], B - 1)
    dst1d = jnp.minimum(edge_index[1], B - 1)
    degp = _sc_segsum(ones, src1d, dst1d)
    feats2d = _conv_feats(data3, cw, cb)
    cur1_2d = _matvec(feats2d, W1, b1.reshape(1, B))

    p1 = _sc_segsum(cur1_2d.reshape(B), src1d, dst1d)
    cur2_2d, cur3_2d = _mv2(cur1_2d, p1, degp, W2, b2.reshape(1, B))

    p2 = _sc_segsum(cur3_2d.reshape(B), src1d, dst1d)
    cur4_2d, out5 = _final(cur3_2d, p2, degp, W3.reshape(1, B),
                           b3.reshape(1, 1))

    return (feats2d.reshape(B), cur1_2d.reshape(B), cur2_2d.reshape(B),
            cur3_2d.reshape(B), cur4_2d.reshape(B), out5.reshape(1))


# speedup vs baseline: 1.6899x; 1.4048x over previous
import functools

import jax
import jax.numpy as jnp
from jax import lax
from jax.experimental import pallas as pl
from jax.experimental.pallas import tpu as pltpu
from jax.experimental.pallas import tpu_sc as plsc

B = 4096
E = 131072
NC = 2
NS = 16
L = 16
NW = NC * NS
EPW = E // NW
ROWS_PW = EPW // 128


_CONV_BLK = 256


def _conv_body(cw_ref, cb_ref, x_ref, out_ref):
    w00 = cw_ref[0, 0]
    w01 = cw_ref[0, 1]
    w10 = cw_ref[0, 2]
    w11 = cw_ref[0, 3]
    xa = x_ref[0:63, :, :]
    xb = x_ref[1:64, :, :]
    a = w00 * xa + w10 * xb + cb_ref[0, 0]
    b = w01 * xa + w11 * xb
    bs = pltpu.roll(b, 63, 1)
    m = a + bs
    s = 1.0 / (1.0 + jnp.exp2(m))
    mx = (lax.broadcasted_iota(jnp.int32, (1, 64, 1), 1) < 63).astype(
        jnp.float32)
    sm = s * mx
    t = jnp.sum(sm, axis=(0, 1))
    out_ref[...] = (t * (1.0 / 3969.0)).reshape(1, _CONV_BLK)


def _conv_feats(data_t, cw, cb):
    nblk = B // _CONV_BLK
    return pl.pallas_call(
        _conv_body,
        grid=(nblk,),
        in_specs=[
            pl.BlockSpec(memory_space=pltpu.SMEM),
            pl.BlockSpec(memory_space=pltpu.SMEM),
            pl.BlockSpec((64, 64, _CONV_BLK), lambda i: (0, 0, i)),
        ],
        out_specs=pl.BlockSpec((1, _CONV_BLK), lambda i: (0, i)),
        out_shape=jax.ShapeDtypeStruct((1, B), jnp.float32),
    )(cw, cb, data_t)


_MV_BLK = 512


def _mv_body(x_ref, w_ref, b_ref, out_ref):
    t = jnp.sum(w_ref[...] * x_ref[...], axis=1)
    out_ref[...] = t.reshape(1, _MV_BLK) + b_ref[...]


def _matvec(x2d, W, b2d):
    nblk = B // _MV_BLK
    return pl.pallas_call(
        _mv_body,
        grid=(nblk,),
        in_specs=[
            pl.BlockSpec((1, B), lambda i: (0, 0)),
            pl.BlockSpec((_MV_BLK, B), lambda i: (i, 0)),
            pl.BlockSpec((1, _MV_BLK), lambda i: (0, i)),
        ],
        out_specs=pl.BlockSpec((1, _MV_BLK), lambda i: (0, i)),
        out_shape=jax.ShapeDtypeStruct((1, B), jnp.float32),
    )(x2d, W, b2d)


def _sc_segsum_body(x_hbm, src_hbm, dst_hbm, out_hbm,
                    x_v, src_v, dst1_v, dst_v, vals_v, zero_v, acc_sh, sem):
    c = lax.axis_index("c")
    s = lax.axis_index("s")
    chunk = c * NS + s

    def zbody(i, _):
        zero_v[pl.ds(i * L, L)] = jnp.zeros((L,), jnp.float32)
        return 0
    lax.fori_loop(0, (B // NS) // L, zbody, 0)
    pltpu.sync_copy(zero_v, acc_sh.at[pl.ds(s * (B // NS), B // NS)])

    pltpu.sync_copy(x_hbm, x_v)
    pltpu.sync_copy(src_hbm.at[pl.ds(chunk * EPW, EPW)], src_v)
    pltpu.sync_copy(dst_hbm.at[pl.ds(chunk * EPW, EPW)], dst1_v)

    def gbody(i, _):
        j = i // (128 // L)
        o = (i % (128 // L)) * L
        idx = src_v[pl.ds(i * L, L)]
        vals_v[j, pl.ds(o, L)] = plsc.load_gather(x_v, [idx])
        dst_v[j, pl.ds(o, L)] = dst1_v[pl.ds(i * L, L)]
        return 0
    lax.fori_loop(0, EPW // L, gbody, 0)

    plsc.subcore_barrier()

    copies = []
    for j in range(ROWS_PW):
        copies.append(
            pltpu.async_copy(vals_v.at[j], acc_sh.at[dst_v.at[j]], sem,
                             add=True))
    for cp in copies:
        cp.wait()

    plsc.subcore_barrier()

    @pl.when(s == 0)
    def _():
        pltpu.sync_copy(acc_sh, out_hbm.at[c])


@functools.lru_cache(maxsize=None)
def _make_sc_segsum():
    mesh = plsc.VectorSubcoreMesh(
        core_axis_name="c", subcore_axis_name="s",
        num_cores=NC, num_subcores=NS)
    return pl.kernel(
        _sc_segsum_body,
        out_type=jax.ShapeDtypeStruct((NC, B), jnp.float32),
        mesh=mesh,
        compiler_params=pltpu.CompilerParams(needs_layout_passes=False),
        scratch_types=[
            pltpu.VMEM((B,), jnp.float32),
            pltpu.VMEM((EPW,), jnp.int32),
            pltpu.VMEM((EPW,), jnp.int32),
            pltpu.VMEM((ROWS_PW, 128), jnp.int32),
            pltpu.VMEM((ROWS_PW, 128), jnp.float32),
            pltpu.VMEM((B // NS,), jnp.float32),
            pltpu.VMEM_SHARED((B,), jnp.float32),
            pltpu.SemaphoreType.DMA,
        ],
    )


def _sc_segsum(x, src1d, dst1d):
    return _make_sc_segsum()(x, src1d, dst1d)


def _nm(p_ref, d_ref):
    sums = p_ref[0:1, :] + p_ref[1:2, :]
    deg = d_ref[0:1, :] + d_ref[1:2, :]
    return jnp.where(deg > 0, sums / jnp.maximum(deg, 1.0), 0.0)


def _mv2_body(x_ref, p_ref, d_ref, w_ref, b_ref, cur2_ref, out_ref):
    cur2 = jnp.tanh(x_ref[...] + _nm(p_ref, d_ref))
    cur2_ref[...] = cur2
    t = jnp.sum(w_ref[...] * cur2, axis=1)
    out_ref[...] = t.reshape(1, _MV_BLK) + b_ref[...]


def _mv2(x2d, p, d, W, b2d):
    nblk = B // _MV_BLK
    return pl.pallas_call(
        _mv2_body,
        grid=(nblk,),
        in_specs=[
            pl.BlockSpec((1, B), lambda i: (0, 0)),
            pl.BlockSpec((NC, B), lambda i: (0, 0)),
            pl.BlockSpec((NC, B), lambda i: (0, 0)),
            pl.BlockSpec((_MV_BLK, B), lambda i: (i, 0)),
            pl.BlockSpec((1, _MV_BLK), lambda i: (0, i)),
        ],
        out_specs=(
            pl.BlockSpec((1, B), lambda i: (0, 0)),
            pl.BlockSpec((1, _MV_BLK), lambda i: (0, i)),
        ),
        out_shape=(
            jax.ShapeDtypeStruct((1, B), jnp.float32),
            jax.ShapeDtypeStruct((1, B), jnp.float32),
        ),
    )(x2d, p, d, W, b2d)


def _final_body(b3_ref, x_ref, p_ref, d_ref, w3_ref, cur4_ref, out_ref):
    cur4 = jnp.tanh(x_ref[...] + _nm(p_ref, d_ref))
    cur4_ref[...] = cur4
    out_ref[0, 0] = jnp.sum(cur4 * w3_ref[...]) + b3_ref[0, 0]


def _final(x2d, p, d, W3, b3_2d):
    return pl.pallas_call(
        _final_body,
        in_specs=[
            pl.BlockSpec(memory_space=pltpu.SMEM),
            pl.BlockSpec((1, B), lambda: (0, 0)),
            pl.BlockSpec((NC, B), lambda: (0, 0)),
            pl.BlockSpec((NC, B), lambda: (0, 0)),
            pl.BlockSpec((1, B), lambda: (0, 0)),
        ],
        out_specs=(
            pl.BlockSpec((1, B), lambda: (0, 0)),
            pl.BlockSpec(memory_space=pltpu.SMEM),
        ),
        out_shape=(
            jax.ShapeDtypeStruct((1, B), jnp.float32),
            jax.ShapeDtypeStruct((1, 1), jnp.float32),
        ),
    )(b3_2d, x2d, p, d, W3)


def kernel(data, edge_index, conv_w, conv_b, W1, b1, W2, b2, W3, b3):
    data_t = jnp.transpose(data, (1, 2, 3, 0)).reshape(64, 64, B)
    neg_log2e = -1.4426950408889634
    cw = conv_w.reshape(1, 4) * neg_log2e
    cb = conv_b.reshape(1, 1) * neg_log2e
    ones = jnp.ones((B,), jnp.float32)

    src1d = jnp.minimum(edge_index[0], B - 1)
    dst1d = jnp.minimum(edge_index[1], B - 1)
    degp = _sc_segsum(ones, src1d, dst1d)
    feats2d = _conv_feats(data_t, cw, cb)
    cur1_2d = _matvec(feats2d, W1, b1.reshape(1, B))

    p1 = _sc_segsum(cur1_2d.reshape(B), src1d, dst1d)
    cur2_2d, cur3_2d = _mv2(cur1_2d, p1, degp, W2, b2.reshape(1, B))

    p2 = _sc_segsum(cur3_2d.reshape(B), src1d, dst1d)
    cur4_2d, out5 = _final(cur3_2d, p2, degp, W3.reshape(1, B),
                           b3.reshape(1, 1))

    return (feats2d.reshape(B), cur1_2d.reshape(B), cur2_2d.reshape(B),
            cur3_2d.reshape(B), cur4_2d.reshape(B), out5.reshape(1))


# speedup vs baseline: 1.7189x; 1.0171x over previous
import functools

import jax
import jax.numpy as jnp
from jax import lax
from jax.experimental import pallas as pl
from jax.experimental.pallas import tpu as pltpu
from jax.experimental.pallas import tpu_sc as plsc

B = 4096
E = 131072
NC = 2
NS = 16
L = 16
NW = NC * NS
EPW = E // NW
ROWS_PW = EPW // 128


_CONV_BLK = 256


def _conv_body(cw_ref, cb_ref, x_ref, out_ref):
    w00 = cw_ref[0, 0]
    w01 = cw_ref[0, 1]
    w10 = cw_ref[0, 2]
    w11 = cw_ref[0, 3]
    xa = x_ref[0:63, :, :]
    xb = x_ref[1:64, :, :]
    a = w00 * xa + w10 * xb + cb_ref[0, 0]
    b = w01 * xa + w11 * xb
    bs = pltpu.roll(b, 63, 1)
    th = jnp.tanh(a + bs)
    mx = (lax.broadcasted_iota(jnp.int32, (1, 64, 1), 1) < 63).astype(
        jnp.float32)
    t = jnp.sum(th * mx, axis=(0, 1))
    out_ref[...] = (t * (0.5 / 3969.0) + 0.5).reshape(1, _CONV_BLK)


def _conv_feats(data_t, cw, cb):
    nblk = B // _CONV_BLK
    return pl.pallas_call(
        _conv_body,
        grid=(nblk,),
        in_specs=[
            pl.BlockSpec(memory_space=pltpu.SMEM),
            pl.BlockSpec(memory_space=pltpu.SMEM),
            pl.BlockSpec((64, 64, _CONV_BLK), lambda i: (0, 0, i)),
        ],
        out_specs=pl.BlockSpec((1, _CONV_BLK), lambda i: (0, i)),
        out_shape=jax.ShapeDtypeStruct((1, B), jnp.float32),
    )(cw, cb, data_t)


_MV_BLK = 512


def _mv_body(x_ref, w_ref, b_ref, out_ref):
    t = jnp.sum(w_ref[...] * x_ref[...], axis=1)
    out_ref[...] = t.reshape(1, _MV_BLK) + b_ref[...]


def _matvec(x2d, W, b2d):
    nblk = B // _MV_BLK
    return pl.pallas_call(
        _mv_body,
        grid=(nblk,),
        in_specs=[
            pl.BlockSpec((1, B), lambda i: (0, 0)),
            pl.BlockSpec((_MV_BLK, B), lambda i: (i, 0)),
            pl.BlockSpec((1, _MV_BLK), lambda i: (0, i)),
        ],
        out_specs=pl.BlockSpec((1, _MV_BLK), lambda i: (0, i)),
        out_shape=jax.ShapeDtypeStruct((1, B), jnp.float32),
    )(x2d, W, b2d)


def _sc_segsum_body(x_hbm, src_hbm, dst_hbm, out_hbm,
                    x_v, src_v, dst1_v, dst_v, vals_v, zero_v, acc_sh, sem):
    c = lax.axis_index("c")
    s = lax.axis_index("s")
    chunk = c * NS + s

    def zbody(i, _):
        zero_v[pl.ds(i * L, L)] = jnp.zeros((L,), jnp.float32)
        return 0
    lax.fori_loop(0, (B // NS) // L, zbody, 0)
    pltpu.sync_copy(zero_v, acc_sh.at[pl.ds(s * (B // NS), B // NS)])

    pltpu.sync_copy(x_hbm, x_v)
    pltpu.sync_copy(src_hbm.at[pl.ds(chunk * EPW, EPW)], src_v)
    pltpu.sync_copy(dst_hbm.at[pl.ds(chunk * EPW, EPW)], dst1_v)

    plsc.subcore_barrier()

    copies = []
    for j in range(ROWS_PW):
        for k in range(128 // L):
            i = j * (128 // L) + k
            idx = src_v[pl.ds(i * L, L)]
            vals_v[j, pl.ds(k * L, L)] = plsc.load_gather(x_v, [idx])
            dst_v[j, pl.ds(k * L, L)] = dst1_v[pl.ds(i * L, L)]
        copies.append(
            pltpu.async_copy(vals_v.at[j], acc_sh.at[dst_v.at[j]], sem,
                             add=True))
    for cp in copies:
        cp.wait()

    plsc.subcore_barrier()

    @pl.when(s == 0)
    def _():
        pltpu.sync_copy(acc_sh, out_hbm.at[c])


@functools.lru_cache(maxsize=None)
def _make_sc_segsum():
    mesh = plsc.VectorSubcoreMesh(
        core_axis_name="c", subcore_axis_name="s",
        num_cores=NC, num_subcores=NS)
    return pl.kernel(
        _sc_segsum_body,
        out_type=jax.ShapeDtypeStruct((NC, B), jnp.float32),
        mesh=mesh,
        compiler_params=pltpu.CompilerParams(needs_layout_passes=False),
        scratch_types=[
            pltpu.VMEM((B,), jnp.float32),
            pltpu.VMEM((EPW,), jnp.int32),
            pltpu.VMEM((EPW,), jnp.int32),
            pltpu.VMEM((ROWS_PW, 128), jnp.int32),
            pltpu.VMEM((ROWS_PW, 128), jnp.float32),
            pltpu.VMEM((B // NS,), jnp.float32),
            pltpu.VMEM_SHARED((B,), jnp.float32),
            pltpu.SemaphoreType.DMA,
        ],
    )


def _sc_segsum(x, src1d, dst1d):
    return _make_sc_segsum()(x, src1d, dst1d)


def _nm(p_ref, d_ref):
    sums = p_ref[0:1, :] + p_ref[1:2, :]
    deg = d_ref[0:1, :] + d_ref[1:2, :]
    return jnp.where(deg > 0, sums / jnp.maximum(deg, 1.0), 0.0)


def _mv2_body(x_ref, p_ref, d_ref, w_ref, b_ref, cur2_ref, out_ref):
    cur2 = jnp.tanh(x_ref[...] + _nm(p_ref, d_ref))
    cur2_ref[...] = cur2
    t = jnp.sum(w_ref[...] * cur2, axis=1)
    out_ref[...] = t.reshape(1, _MV_BLK) + b_ref[...]


def _mv2(x2d, p, d, W, b2d):
    nblk = B // _MV_BLK
    return pl.pallas_call(
        _mv2_body,
        grid=(nblk,),
        in_specs=[
            pl.BlockSpec((1, B), lambda i: (0, 0)),
            pl.BlockSpec((NC, B), lambda i: (0, 0)),
            pl.BlockSpec((NC, B), lambda i: (0, 0)),
            pl.BlockSpec((_MV_BLK, B), lambda i: (i, 0)),
            pl.BlockSpec((1, _MV_BLK), lambda i: (0, i)),
        ],
        out_specs=(
            pl.BlockSpec((1, B), lambda i: (0, 0)),
            pl.BlockSpec((1, _MV_BLK), lambda i: (0, i)),
        ),
        out_shape=(
            jax.ShapeDtypeStruct((1, B), jnp.float32),
            jax.ShapeDtypeStruct((1, B), jnp.float32),
        ),
    )(x2d, p, d, W, b2d)


def _final_body(b3_ref, x_ref, p_ref, d_ref, w3_ref, cur4_ref, out_ref):
    cur4 = jnp.tanh(x_ref[...] + _nm(p_ref, d_ref))
    cur4_ref[...] = cur4
    out_ref[0, 0] = jnp.sum(cur4 * w3_ref[...]) + b3_ref[0, 0]


def _final(x2d, p, d, W3, b3_2d):
    return pl.pallas_call(
        _final_body,
        in_specs=[
            pl.BlockSpec(memory_space=pltpu.SMEM),
            pl.BlockSpec((1, B), lambda: (0, 0)),
            pl.BlockSpec((NC, B), lambda: (0, 0)),
            pl.BlockSpec((NC, B), lambda: (0, 0)),
            pl.BlockSpec((1, B), lambda: (0, 0)),
        ],
        out_specs=(
            pl.BlockSpec((1, B), lambda: (0, 0)),
            pl.BlockSpec(memory_space=pltpu.SMEM),
        ),
        out_shape=(
            jax.ShapeDtypeStruct((1, B), jnp.float32),
            jax.ShapeDtypeStruct((1, 1), jnp.float32),
        ),
    )(b3_2d, x2d, p, d, W3)


def kernel(data, edge_index, conv_w, conv_b, W1, b1, W2, b2, W3, b3):
    data_t = jnp.transpose(data, (1, 2, 3, 0)).reshape(64, 64, B)
    cw = conv_w.reshape(1, 4) * 0.5
    cb = conv_b.reshape(1, 1) * 0.5
    ones = jnp.ones((B,), jnp.float32)

    src1d = jnp.minimum(edge_index[0], B - 1)
    dst1d = jnp.minimum(edge_index[1], B - 1)
    degp = _sc_segsum(ones, src1d, dst1d)
    feats2d = _conv_feats(data_t, cw, cb)
    cur1_2d = _matvec(feats2d, W1, b1.reshape(1, B))

    p1 = _sc_segsum(cur1_2d.reshape(B), src1d, dst1d)
    cur2_2d, cur3_2d = _mv2(cur1_2d, p1, degp, W2, b2.reshape(1, B))

    p2 = _sc_segsum(cur3_2d.reshape(B), src1d, dst1d)
    cur4_2d, out5 = _final(cur3_2d, p2, degp, W3.reshape(1, B),
                           b3.reshape(1, 1))

    return (feats2d.reshape(B), cur1_2d.reshape(B), cur2_2d.reshape(B),
            cur3_2d.reshape(B), cur4_2d.reshape(B), out5.reshape(1))
